# baseline (speedup 1.0000x reference)
import functools

import jax
import jax.numpy as jnp
from jax.experimental import pallas as pl
from jax.experimental.pallas import tpu as pltpu
from jax.experimental.pallas import tpu_sc as plsc

K = 20
C = 256
W = 2
L = 16


def _sc_body(f_v, x_v, o_v):
    @pl.loop(0, W)
    def _(r):
        @pl.loop(0, C // L)
        def _(ci):
            c0 = ci * L
            cv = f_v[0, r, pl.ds(c0, L)]
            for k in range(K):
                xv = x_v[k, r, pl.ds(c0, L)]
                o_v[k, r, pl.ds(c0, L)] = cv
                o_v[k, r, pl.ds(C + c0, L)] = xv - cv


def kernel(fts, knn_fts):
    B, N, _ = fts.shape
    knn_t = jnp.transpose(knn_fts, (0, 2, 1, 3))
    x3 = knn_t.reshape(B * K, N, C)

    mesh = plsc.VectorSubcoreMesh(core_axis_name="c", subcore_axis_name="s")

    @functools.partial(
        pl.kernel,
        out_type=jax.ShapeDtypeStruct((B * K, N, 2 * C), fts.dtype),
        mesh=mesh,
    )
    def sck(f_hbm, x_hbm, o_hbm):
        pltpu.emit_pipeline(
            _sc_body,
            grid=(B, N // W),
            in_specs=[
                pl.BlockSpec((1, W, C), lambda b, j: (b, j, 0)),
                pl.BlockSpec((K, W, C), lambda b, j: (b, j, 0)),
            ],
            out_specs=[pl.BlockSpec((K, W, 2 * C), lambda b, j: (b, j, 0))],
            core_axis_name=("c", "s"),
            dimension_semantics=(pltpu.PARALLEL, pltpu.PARALLEL),
        )(f_hbm, x_hbm, o_hbm)

    out3 = sck(fts, x3)
    out_t = out3.reshape(B, K, N, 2 * C)
    return jnp.transpose(out_t, (0, 2, 1, 3))
